# baseline (speedup 1.0000x reference)
import jax
import jax.numpy as jnp
from jax.experimental import pallas as pl

_MARGIN = 1.0
_B = 1024
_NPAIRS = _B * (_B - 1) // 2


def _loss_kernel(e_ref, t_ref, out_ref):
    e = e_ref[...]
    t = t_ref[...]
    gram = jax.lax.dot_general(
        e, e, (((1,), (1,)), ((), ())),
        preferred_element_type=jnp.float32)
    sq = jnp.sum(e * e, axis=1, keepdims=True)
    sqdist = jnp.maximum(sq + sq.T - 2.0 * gram, 0.0)

    dist = jnp.sqrt(jnp.maximum(sqdist, 1e-12))
    neg = jnp.maximum(_MARGIN - dist, 0.0)
    neg = neg * neg

    eq = t == t.T
    loss = jnp.where(eq, sqdist, neg)

    total = jnp.sum(loss)
    out_ref[...] = (total * (0.5 / _NPAIRS)).reshape(1, 1)


def kernel(embeddings, target):
    t = target.astype(jnp.int32).reshape(_B, 1)
    out = pl.pallas_call(
        _loss_kernel,
        out_shape=jax.ShapeDtypeStruct((1, 1), jnp.float32),
    )(embeddings, t)
    return out[0, 0]


# speedup vs baseline: 1.0020x; 1.0020x over previous
import jax
import jax.numpy as jnp
from jax.experimental import pallas as pl

_MARGIN = 1.0
_B = 1024
_NPAIRS = _B * (_B - 1) // 2


def _loss_kernel(e_ref, t_ref, out_ref):
    e = e_ref[...]
    t = t_ref[...]
    gram = jnp.dot(e, e.T, preferred_element_type=jnp.float32)
    sq = jnp.sum(e * e, axis=1, keepdims=True)
    sqdist = jnp.maximum(sq + sq.T - 2.0 * gram, 0.0)

    dist = jnp.sqrt(jnp.maximum(sqdist, 1e-12))
    neg = jnp.maximum(_MARGIN - dist, 0.0)
    neg = neg * neg

    eq = t == t.T
    loss = jnp.where(eq, sqdist, neg)

    total = jnp.sum(loss)
    out_ref[...] = (total * (0.5 / _NPAIRS)).reshape(1, 1)


def kernel(embeddings, target):
    t = target.astype(jnp.int32).reshape(_B, 1)
    out = pl.pallas_call(
        _loss_kernel,
        out_shape=jax.ShapeDtypeStruct((1, 1), jnp.float32),
    )(embeddings, t)
    return out[0, 0]


# speedup vs baseline: 1.1791x; 1.1768x over previous
import jax
import jax.numpy as jnp
from jax.experimental import pallas as pl

_MARGIN = 1.0
_B = 1024
_NPAIRS = _B * (_B - 1) // 2


def _loss_kernel(e_ref, t_ref, out_ref):
    e = e_ref[...]
    t = t_ref[...]
    gram = jnp.dot(e, e.T, preferred_element_type=jnp.float32)
    sq = jnp.sum(e * e, axis=1, keepdims=True)
    sqdist = jnp.maximum(sq + sq.T - 2.0 * gram, 0.0)

    dist = jnp.sqrt(jnp.maximum(sqdist, 1e-12))
    neg = jnp.maximum(_MARGIN - dist, 0.0)
    neg = neg * neg

    eq = t == t.T
    loss = jnp.where(eq, sqdist, neg)

    rows = jax.lax.broadcasted_iota(jnp.int32, (_B, _B), 0)
    cols = jax.lax.broadcasted_iota(jnp.int32, (_B, _B), 1)
    total = jnp.sum(jnp.where(cols > rows, loss, 0.0))
    out_ref[...] = (total * (1.0 / _NPAIRS)).reshape(1, 1)


def kernel(embeddings, target):
    t = target.astype(jnp.int32).reshape(_B, 1)
    out = pl.pallas_call(
        _loss_kernel,
        out_shape=jax.ShapeDtypeStruct((1, 1), jnp.float32),
    )(embeddings, t)
    return out[0, 0]


# speedup vs baseline: 1.6901x; 1.4334x over previous
import jax
import jax.numpy as jnp
from jax.experimental import pallas as pl

_MARGIN = 1.0
_B = 1024
_NPAIRS = _B * (_B - 1) // 2


def _loss_kernel(e_ref, t_ref, out_ref):
    e = e_ref[...]
    t = t_ref[...]
    gram = jnp.dot(e, e.T, preferred_element_type=jnp.float32)
    sq = jnp.sum(e * e, axis=1, keepdims=True)
    sqdist = jnp.maximum(sq + sq.T - 2.0 * gram, 0.0)

    dist = jnp.sqrt(jnp.maximum(sqdist, 1e-12))
    neg = jnp.maximum(_MARGIN - dist, 0.0)
    neg = neg * neg

    eq = t.T == t
    loss = jnp.where(eq, sqdist, neg)

    rows = jax.lax.broadcasted_iota(jnp.int32, (_B, _B), 0)
    cols = jax.lax.broadcasted_iota(jnp.int32, (_B, _B), 1)
    total = jnp.sum(jnp.where(cols > rows, loss, 0.0))
    out_ref[...] = (total * (1.0 / _NPAIRS)).reshape(1, 1)


def kernel(embeddings, target):
    t = target.astype(jnp.int32).reshape(1, _B)
    out = pl.pallas_call(
        _loss_kernel,
        out_shape=jax.ShapeDtypeStruct((1, 1), jnp.float32),
    )(embeddings, t)
    return out.reshape(())


# speedup vs baseline: 1.9291x; 1.1414x over previous
import jax
import jax.numpy as jnp
from jax.experimental import pallas as pl

_MARGIN = 1.0
_B = 1024
_NPAIRS = _B * (_B - 1) // 2


def _loss_kernel(e_ref, t_ref, out_ref):
    e = e_ref[...]
    t = t_ref[...]
    gram = jnp.dot(e, e.T, preferred_element_type=jnp.float32)
    sq = jnp.sum(e * e, axis=1, keepdims=True)
    sqdist = jnp.maximum(sq + sq.T - 2.0 * gram, 1e-12)

    dist = sqdist * jax.lax.rsqrt(sqdist)
    neg = jnp.maximum(_MARGIN - dist, 0.0)
    neg = neg * neg

    eq = t.T == t
    loss = jnp.where(eq, sqdist, neg)

    rows = jax.lax.broadcasted_iota(jnp.int32, (_B, _B), 0)
    cols = jax.lax.broadcasted_iota(jnp.int32, (_B, _B), 1)
    total = jnp.sum(jnp.where(cols > rows, loss, 0.0))
    out_ref[...] = (total * (1.0 / _NPAIRS)).reshape(1, 1)


def kernel(embeddings, target):
    t = target.astype(jnp.int32).reshape(1, _B)
    out = pl.pallas_call(
        _loss_kernel,
        out_shape=jax.ShapeDtypeStruct((1, 1), jnp.float32),
    )(embeddings, t)
    return out.reshape(())
